# baseline (speedup 1.0000x reference)
import functools

import jax
import jax.numpy as jnp
from jax import lax
from jax.experimental import pallas as pl
from jax.experimental.pallas import tpu as pltpu
from jax.experimental.pallas import tpu_sc as plsc

_B, _S, _D = 4, 8192, 768
_NC, _NS = 2, 16
_NW = _NC * _NS
_ROWS_W = _S // _NW
_CH = 64
_CHUNKS = _ROWS_W // _CH

_mesh = plsc.VectorSubcoreMesh(core_axis_name="c", subcore_axis_name="s")


@functools.partial(
    pl.kernel,
    mesh=_mesh,
    out_type=jax.ShapeDtypeStruct((_B, _S, _D), jnp.float32),
    scratch_types=[
        pltpu.VMEM((2, _CH, _D), jnp.float32),
        pltpu.SemaphoreType.DMA,
        pltpu.SemaphoreType.DMA,
        pltpu.SemaphoreType.DMA,
    ],
)
def _sc_broadcast_copy(table_hbm, out_hbm, buf, sem_r, sem_w0, sem_w1):
    wid = lax.axis_index("s") * _NC + lax.axis_index("c")
    base = wid * _ROWS_W
    wsems = (sem_w0, sem_w1)
    writes = [[], []]

    rd = pltpu.async_copy(table_hbm.at[pl.ds(base, _CH)], buf.at[0], sem_r)
    for i in range(_CHUNKS):
        sl = i % 2
        nsl = (i + 1) % 2
        rd.wait()
        if i + 1 < _CHUNKS:
            for w in writes[nsl]:
                w.wait()
            writes[nsl] = []
            rd = pltpu.async_copy(
                table_hbm.at[pl.ds(base + (i + 1) * _CH, _CH)], buf.at[nsl], sem_r
            )
        r0 = base + i * _CH
        for b in range(_B):
            writes[sl].append(
                pltpu.async_copy(buf.at[sl], out_hbm.at[b, pl.ds(r0, _CH)], wsems[sl])
            )
    for sl in range(2):
        for w in writes[sl]:
            w.wait()


def kernel(x, pos_embed_weight):
    del x
    return _sc_broadcast_copy(pos_embed_weight)
